# baseline (speedup 1.0000x reference)
import jax
import jax.numpy as jnp
from jax.experimental import pallas as pl

TH = 0.85
BI = 512
NB = 8


def _qlayer_kern(x_ref, w_ref, b_ref, out_ref):
    f = x_ref.shape[1]
    xa = x_ref[:]
    ones_row = jnp.ones((1, f), dtype=jnp.float32)
    ones_col = jnp.ones((f, 1), dtype=jnp.float32)
    rxs_row = jnp.dot(ones_row, xa.T, preferred_element_type=jnp.float32)
    rxs_col = jnp.dot(xa, ones_col, preferred_element_type=jnp.float32)
    wq = w_ref[:].astype(jnp.bfloat16).astype(jnp.float32)
    wsum = jnp.sum(wq, axis=1)[None, :]

    xblk = [xa[bi * BI:(bi + 1) * BI, :] for bi in range(NB)]
    rrow = [rxs_row[:, bi * BI:(bi + 1) * BI] for bi in range(NB)]
    rcol = [rxs_col[bi * BI:(bi + 1) * BI, :] for bi in range(NB)]

    t = [None] * NB
    tcol = [None] * NB
    for bi in range(NB):
        xb = xblk[bi]
        gram = jnp.dot(xb, xb.T, preferred_element_type=jnp.float32)
        c = gram * gram >= TH
        acc = jnp.sum(jnp.where(c, rrow[bi], 0.0), axis=1, keepdims=True)
        sq = jnp.dot(xb * xb, ones_col, preferred_element_type=jnp.float32)
        diag_c = (sq * sq >= TH).astype(jnp.float32)
        t[bi] = acc - diag_c * rcol[bi]
        for bj in range(bi + 1, NB):
            gram = jnp.dot(xb, xblk[bj].T, preferred_element_type=jnp.float32)
            c = gram * gram >= TH
            t[bi] = t[bi] + jnp.sum(jnp.where(c, rrow[bj], 0.0),
                                    axis=1, keepdims=True)
            cr = jnp.sum(jnp.where(c, rcol[bi], 0.0), axis=0, keepdims=True)
            tcol[bj] = cr if tcol[bj] is None else tcol[bj] + cr

    for bi in range(NB):
        tt = t[bi] if tcol[bi] is None else t[bi] + tcol[bi].reshape(BI, 1)
        scalar = (tt / f).astype(jnp.bfloat16).astype(jnp.float32)
        out_ref[bi * BI:(bi + 1) * BI, :] = scalar * wsum + b_ref[:]


@jax.jit
def kernel(x, W, b):
    n, f = x.shape
    h = W.shape[0]
    b2 = b.reshape(1, h)
    return pl.pallas_call(
        _qlayer_kern,
        out_shape=jax.ShapeDtypeStruct((n, h), jnp.float32),
    )(x, W, b2)


# speedup vs baseline: 1.1038x; 1.1038x over previous
import jax
import jax.numpy as jnp
from jax.experimental import pallas as pl

TH = 0.85
BI = 1024
NB = 4


def _qlayer_kern(x_ref, w_ref, b_ref, out_ref):
    f = x_ref.shape[1]
    xa = x_ref[:]
    ones_row = jnp.ones((1, f), dtype=jnp.float32)
    ones_col = jnp.ones((f, 1), dtype=jnp.float32)
    rxs_row = jnp.dot(ones_row, xa.T, preferred_element_type=jnp.float32)
    rxs_col = jnp.dot(xa, ones_col, preferred_element_type=jnp.float32)
    wq = w_ref[:].astype(jnp.bfloat16).astype(jnp.float32)
    wsum = jnp.sum(wq, axis=1)[None, :]

    xblk = [xa[bi * BI:(bi + 1) * BI, :] for bi in range(NB)]
    rrow = [rxs_row[:, bi * BI:(bi + 1) * BI] for bi in range(NB)]
    rcol = [rxs_col[bi * BI:(bi + 1) * BI, :] for bi in range(NB)]

    t = [None] * NB
    tcol = [None] * NB
    for bi in range(NB):
        xb = xblk[bi]
        gram = jnp.dot(xb, xb.T, preferred_element_type=jnp.float32)
        c = gram * gram >= TH
        acc = jnp.sum(jnp.where(c, rrow[bi], 0.0), axis=1, keepdims=True)
        sq = jnp.dot(xb * xb, ones_col, preferred_element_type=jnp.float32)
        diag_c = (sq * sq >= TH).astype(jnp.float32)
        t[bi] = acc - diag_c * rcol[bi]
        for bj in range(bi + 1, NB):
            gram = jnp.dot(xb, xblk[bj].T, preferred_element_type=jnp.float32)
            c = gram * gram >= TH
            t[bi] = t[bi] + jnp.sum(jnp.where(c, rrow[bj], 0.0),
                                    axis=1, keepdims=True)
            cr = jnp.sum(jnp.where(c, rcol[bi], 0.0), axis=0, keepdims=True)
            tcol[bj] = cr if tcol[bj] is None else tcol[bj] + cr

    for bi in range(NB):
        tt = t[bi] if tcol[bi] is None else t[bi] + tcol[bi].reshape(BI, 1)
        scalar = (tt / f).astype(jnp.bfloat16).astype(jnp.float32)
        out_ref[bi * BI:(bi + 1) * BI, :] = scalar * wsum + b_ref[:]


@jax.jit
def kernel(x, W, b):
    n, f = x.shape
    h = W.shape[0]
    b2 = b.reshape(1, h)
    return pl.pallas_call(
        _qlayer_kern,
        out_shape=jax.ShapeDtypeStruct((n, h), jnp.float32),
    )(x, W, b2)
